# baseline (speedup 1.0000x reference)
import functools

import jax
import jax.numpy as jnp
from jax import lax
from jax.experimental import pallas as pl
from jax.experimental.pallas import tpu as pltpu
from jax.experimental.pallas import tpu_sc as plsc

N = 10000
E = 320000
D = 128
EPS = 1e-5

NUM_CORES = 2
NUM_SUBCORES = 16
NW = NUM_CORES * NUM_SUBCORES
CHUNK = 80
CHUNKS_PER_W = 125
HALF_SIZES = (64, 61)
ROWS_PER_TILE = 632
NPAD = ROWS_PER_TILE * NUM_SUBCORES


def _sc_scatter_kernel(h_hbm, src_hbm, dst_hbm, zero_hbm, out_hbm,
                       idx_s, idx_d, rows_a, rows_b, rows_c, acc,
                       sem_ga, sem_gb, sem_gc, sem_sa, sem_sb, sem_sc,
                       sem_z):
    c = lax.axis_index("c")
    s = lax.axis_index("s")
    wid = s * NUM_CORES + c

    zsl = pl.ds(s * ROWS_PER_TILE, ROWS_PER_TILE)
    pltpu.async_copy(zero_hbm, acc.at[zsl], sem_z)

    hstart = 0
    for hsize in HALF_SIZES:
        ntrip = hsize // 3
        ec = 3 * ntrip

        def body(j, carry, ntrip=ntrip):
            c0 = 3 * j
            c1 = c0 + 1
            c2 = c0 + 2
            pltpu.make_async_copy(h_hbm.at[idx_s.at[c0]], rows_a, sem_ga).wait()
            pltpu.async_copy(h_hbm.at[idx_s.at[c2]], rows_c, sem_gc)
            pltpu.async_copy(rows_a, acc.at[idx_d.at[c0]], sem_sa, add=True)
            pltpu.make_async_copy(h_hbm.at[idx_s.at[c1]], rows_b, sem_gb).wait()
            pltpu.async_copy(rows_b, acc.at[idx_d.at[c1]], sem_sb, add=True)
            pltpu.make_async_copy(rows_a, acc.at[idx_d.at[c0]], sem_sa).wait()

            pltpu.async_copy(h_hbm.at[idx_s.at[c0 + 3]], rows_a, sem_ga)

            pltpu.make_async_copy(h_hbm.at[idx_s.at[c2]], rows_c, sem_gc).wait()
            pltpu.async_copy(rows_c, acc.at[idx_d.at[c2]], sem_sc, add=True)
            pltpu.make_async_copy(rows_b, acc.at[idx_d.at[c1]], sem_sb).wait()

            @pl.when(j < ntrip - 1)
            def _():
                pltpu.async_copy(h_hbm.at[idx_s.at[c0 + 4]], rows_b, sem_gb)

            pltpu.make_async_copy(rows_c, acc.at[idx_d.at[c2]], sem_sc).wait()
            return carry

        sl = pl.ds(0, hsize)
        pltpu.sync_copy(src_hbm.at[wid, pl.ds(hstart, hsize)], idx_s.at[sl])
        pltpu.sync_copy(dst_hbm.at[wid, pl.ds(hstart, hsize)], idx_d.at[sl])
        pltpu.async_copy(h_hbm.at[idx_s.at[0]], rows_a, sem_ga)
        pltpu.async_copy(h_hbm.at[idx_s.at[1]], rows_b, sem_gb)
        if hstart == 0:
            pltpu.make_async_copy(zero_hbm, acc.at[zsl], sem_z).wait()
            plsc.subcore_barrier()
        lax.fori_loop(0, ntrip, body, 0)
        pltpu.make_async_copy(h_hbm.at[idx_s.at[ec]], rows_a, sem_ga).wait()
        pltpu.async_copy(rows_a, acc.at[idx_d.at[ec]], sem_sa, add=True)
        pltpu.make_async_copy(rows_a, acc.at[idx_d.at[ec]], sem_sa).wait()
        hstart += hsize

    plsc.subcore_barrier()

    sl = pl.ds(s * ROWS_PER_TILE, ROWS_PER_TILE)
    pltpu.sync_copy(acc.at[sl], out_hbm.at[c, sl])


def _sc_scatter(h, src, dst, zero):
    mesh = plsc.VectorSubcoreMesh(core_axis_name="c", subcore_axis_name="s")
    kfn = pl.kernel(
        _sc_scatter_kernel,
        mesh=mesh,
        out_type=jax.ShapeDtypeStruct((NUM_CORES, NPAD, D), jnp.float32),
        scratch_types=[
            pltpu.VMEM((HALF_SIZES[0], CHUNK), jnp.int32),
            pltpu.VMEM((HALF_SIZES[0], CHUNK), jnp.int32),
            pltpu.VMEM((CHUNK, D), jnp.float32),
            pltpu.VMEM((CHUNK, D), jnp.float32),
            pltpu.VMEM((CHUNK, D), jnp.float32),
            pltpu.VMEM_SHARED((NPAD, D), jnp.float32),
            pltpu.SemaphoreType.DMA,
            pltpu.SemaphoreType.DMA,
            pltpu.SemaphoreType.DMA,
            pltpu.SemaphoreType.DMA,
            pltpu.SemaphoreType.DMA,
            pltpu.SemaphoreType.DMA,
            pltpu.SemaphoreType.DMA,
        ],
    )
    return kfn(h, src, dst, zero)


def _tc_finish_kernel(acc_ref, h_ref, w_ref, b_ref, g_ref, be_ref, o_ref):
    s = acc_ref[0] + acc_ref[1] + h_ref[...]
    x = lax.dot_general(s, w_ref[...], (((1,), (1,)), ((), ())),
                        preferred_element_type=jnp.float32)
    x = x + b_ref[...]
    mu = jnp.mean(x, axis=1, keepdims=True)
    xc = x - mu
    var = jnp.mean(xc * xc, axis=1, keepdims=True)
    y = xc * lax.rsqrt(var + EPS) * g_ref[...] + be_ref[...]
    o_ref[...] = jnp.maximum(y, 0.0)


def _tc_finish(accp, h, W, b, gamma, beta):
    blk = 2000
    grid = (N // blk,)
    return pl.pallas_call(
        _tc_finish_kernel,
        grid=grid,
        in_specs=[
            pl.BlockSpec((NUM_CORES, blk, D), lambda i: (0, i, 0)),
            pl.BlockSpec((blk, D), lambda i: (i, 0)),
            pl.BlockSpec((D, D), lambda i: (0, 0)),
            pl.BlockSpec((1, D), lambda i: (0, 0)),
            pl.BlockSpec((1, D), lambda i: (0, 0)),
            pl.BlockSpec((1, D), lambda i: (0, 0)),
        ],
        out_specs=pl.BlockSpec((blk, D), lambda i: (i, 0)),
        out_shape=jax.ShapeDtypeStruct((N, D), jnp.float32),
    )(accp, h, W, b, gamma, beta)


def kernel(h, edge_index, W, b, gamma, beta):
    src = edge_index[0].reshape(NW, CHUNKS_PER_W, CHUNK)
    dst = edge_index[1].reshape(NW, CHUNKS_PER_W, CHUNK)
    zero = jnp.zeros((ROWS_PER_TILE, D), jnp.float32)
    accp = _sc_scatter(h, src, dst, zero)
    return _tc_finish(accp, h, W.astype(jnp.float32),
                      b.reshape(1, D), gamma.reshape(1, D), beta.reshape(1, D))


# speedup vs baseline: 1.0043x; 1.0043x over previous
import functools

import jax
import jax.numpy as jnp
from jax import lax
from jax.experimental import pallas as pl
from jax.experimental.pallas import tpu as pltpu
from jax.experimental.pallas import tpu_sc as plsc

N = 10000
E = 320000
D = 128
EPS = 1e-5

NUM_CORES = 2
NUM_SUBCORES = 16
NW = NUM_CORES * NUM_SUBCORES
CHUNK = 80
CHUNKS_PER_W = 125
HALF_SIZES = (64, 61)
ROWS_PER_TILE = 632
NPAD = ROWS_PER_TILE * NUM_SUBCORES


def _sc_scatter_kernel(h_hbm, src_hbm, dst_hbm, zero_hbm, out_hbm,
                       idx_s, idx_d, rows_a, rows_b, rows_c, acc,
                       sem_ga, sem_gb, sem_gc, sem_sa, sem_sb, sem_sc,
                       sem_z):
    c = lax.axis_index("c")
    s = lax.axis_index("s")
    wid = s * NUM_CORES + c

    zsl = pl.ds(s * ROWS_PER_TILE, ROWS_PER_TILE)
    pltpu.async_copy(zero_hbm, acc.at[zsl], sem_z)

    hstart = 0
    for hsize in HALF_SIZES:
        ntrip = hsize // 3
        ec = 3 * ntrip

        def body(j, carry, ntrip=ntrip):
            c0 = 3 * j
            c1 = c0 + 1
            c2 = c0 + 2
            pltpu.make_async_copy(h_hbm.at[idx_s.at[c0]], rows_a, sem_ga).wait()
            pltpu.async_copy(h_hbm.at[idx_s.at[c2]], rows_c, sem_gc)
            pltpu.async_copy(rows_a, acc.at[idx_d.at[c0]], sem_sa, add=True)
            pltpu.make_async_copy(h_hbm.at[idx_s.at[c1]], rows_b, sem_gb).wait()
            pltpu.make_async_copy(rows_a, acc.at[idx_d.at[c0]], sem_sa).wait()

            pltpu.async_copy(h_hbm.at[idx_s.at[c0 + 3]], rows_a, sem_ga)

            pltpu.async_copy(rows_b, acc.at[idx_d.at[c1]], sem_sb, add=True)
            pltpu.make_async_copy(h_hbm.at[idx_s.at[c2]], rows_c, sem_gc).wait()
            pltpu.make_async_copy(rows_b, acc.at[idx_d.at[c1]], sem_sb).wait()

            @pl.when(j < ntrip - 1)
            def _():
                pltpu.async_copy(h_hbm.at[idx_s.at[c0 + 4]], rows_b, sem_gb)

            pltpu.async_copy(rows_c, acc.at[idx_d.at[c2]], sem_sc, add=True)
            pltpu.make_async_copy(rows_c, acc.at[idx_d.at[c2]], sem_sc).wait()
            return carry

        sl = pl.ds(0, hsize)
        pltpu.sync_copy(src_hbm.at[wid, pl.ds(hstart, hsize)], idx_s.at[sl])
        pltpu.sync_copy(dst_hbm.at[wid, pl.ds(hstart, hsize)], idx_d.at[sl])
        pltpu.async_copy(h_hbm.at[idx_s.at[0]], rows_a, sem_ga)
        pltpu.async_copy(h_hbm.at[idx_s.at[1]], rows_b, sem_gb)
        if hstart == 0:
            pltpu.make_async_copy(zero_hbm, acc.at[zsl], sem_z).wait()
            plsc.subcore_barrier()
        lax.fori_loop(0, ntrip, body, 0)
        pltpu.make_async_copy(h_hbm.at[idx_s.at[ec]], rows_a, sem_ga).wait()
        pltpu.async_copy(rows_a, acc.at[idx_d.at[ec]], sem_sa, add=True)
        pltpu.make_async_copy(rows_a, acc.at[idx_d.at[ec]], sem_sa).wait()
        hstart += hsize

    plsc.subcore_barrier()

    sl = pl.ds(s * ROWS_PER_TILE, ROWS_PER_TILE)
    pltpu.sync_copy(acc.at[sl], out_hbm.at[c, sl])


def _sc_scatter(h, src, dst, zero):
    mesh = plsc.VectorSubcoreMesh(core_axis_name="c", subcore_axis_name="s")
    kfn = pl.kernel(
        _sc_scatter_kernel,
        mesh=mesh,
        out_type=jax.ShapeDtypeStruct((NUM_CORES, NPAD, D), jnp.float32),
        scratch_types=[
            pltpu.VMEM((HALF_SIZES[0], CHUNK), jnp.int32),
            pltpu.VMEM((HALF_SIZES[0], CHUNK), jnp.int32),
            pltpu.VMEM((CHUNK, D), jnp.float32),
            pltpu.VMEM((CHUNK, D), jnp.float32),
            pltpu.VMEM((CHUNK, D), jnp.float32),
            pltpu.VMEM_SHARED((NPAD, D), jnp.float32),
            pltpu.SemaphoreType.DMA,
            pltpu.SemaphoreType.DMA,
            pltpu.SemaphoreType.DMA,
            pltpu.SemaphoreType.DMA,
            pltpu.SemaphoreType.DMA,
            pltpu.SemaphoreType.DMA,
            pltpu.SemaphoreType.DMA,
        ],
    )
    return kfn(h, src, dst, zero)


def _tc_finish_kernel(acc_ref, h_ref, w_ref, b_ref, g_ref, be_ref, o_ref):
    s = acc_ref[0] + acc_ref[1] + h_ref[...]
    x = lax.dot_general(s, w_ref[...], (((1,), (1,)), ((), ())),
                        preferred_element_type=jnp.float32)
    x = x + b_ref[...]
    mu = jnp.mean(x, axis=1, keepdims=True)
    xc = x - mu
    var = jnp.mean(xc * xc, axis=1, keepdims=True)
    y = xc * lax.rsqrt(var + EPS) * g_ref[...] + be_ref[...]
    o_ref[...] = jnp.maximum(y, 0.0)


def _tc_finish(accp, h, W, b, gamma, beta):
    blk = 2000
    grid = (N // blk,)
    return pl.pallas_call(
        _tc_finish_kernel,
        grid=grid,
        in_specs=[
            pl.BlockSpec((NUM_CORES, blk, D), lambda i: (0, i, 0)),
            pl.BlockSpec((blk, D), lambda i: (i, 0)),
            pl.BlockSpec((D, D), lambda i: (0, 0)),
            pl.BlockSpec((1, D), lambda i: (0, 0)),
            pl.BlockSpec((1, D), lambda i: (0, 0)),
            pl.BlockSpec((1, D), lambda i: (0, 0)),
        ],
        out_specs=pl.BlockSpec((blk, D), lambda i: (i, 0)),
        out_shape=jax.ShapeDtypeStruct((N, D), jnp.float32),
    )(accp, h, W, b, gamma, beta)


def kernel(h, edge_index, W, b, gamma, beta):
    src = edge_index[0].reshape(NW, CHUNKS_PER_W, CHUNK)
    dst = edge_index[1].reshape(NW, CHUNKS_PER_W, CHUNK)
    zero = jnp.zeros((ROWS_PER_TILE, D), jnp.float32)
    accp = _sc_scatter(h, src, dst, zero)
    return _tc_finish(accp, h, W.astype(jnp.float32),
                      b.reshape(1, D), gamma.reshape(1, D), beta.reshape(1, D))
